# Initial kernel scaffold; baseline (speedup 1.0000x reference)
import jax
import jax.numpy as jnp
from jax.experimental import pallas as pl


def kernel(x, q_w, q_b, k_w, k_b, v_w, v_b, c_w, c_b, qs_w, qs_b, ks_w, ks_b):
    raise NotImplementedError("write your pallas kernel here")


# speedup vs baseline: 5466.4995x; 5466.4995x over previous
import functools
import math

import jax
import jax.numpy as jnp
from jax import lax
from jax.experimental import pallas as pl
from jax.experimental.pallas import tpu as pltpu
from jax.experimental.pallas import tpu_sc as plsc

N_EMBD = 768
N_HEAD = 12
HS = N_EMBD // N_HEAD
KSEL = 16
T = 2048

TB = 256
NTB = T // TB
PROJ_N = 3 * N_EMBD + 128


def _proj_body(x_ref, w_ref, b_ref, o_ref):
    o_ref[...] = (
        jnp.dot(x_ref[...], w_ref[...], preferred_element_type=jnp.float32)
        + b_ref[...]
    )


def _tc_proj(x2d, w, b):
    return pl.pallas_call(
        _proj_body,
        grid=(NTB,),
        in_specs=[
            pl.BlockSpec((TB, N_EMBD), lambda i: (i, 0)),
            pl.BlockSpec((N_EMBD, PROJ_N), lambda i: (0, 0)),
            pl.BlockSpec((1, PROJ_N), lambda i: (0, 0)),
        ],
        out_specs=pl.BlockSpec((TB, PROJ_N), lambda i: (i, 0)),
        out_shape=jax.ShapeDtypeStruct((T, PROJ_N), jnp.float32),
    )(x2d, w, b)


def _sc_scan_body(scores_hbm, m_hbm, sc_v, m_v, sem):
    cid = lax.axis_index("c")
    sid = lax.axis_index("s")

    @pl.when((cid == 0) & (sid == 0))
    def _():
        pltpu.sync_copy(scores_hbm, sc_v)

        lanes = lax.iota(jnp.int32, 16)

        def init_m(i, carry):
            m_v[pl.ds(pl.multiple_of(i * 16, 16), 16)] = jnp.full(
                (16,), T, jnp.int32
            )
            return carry

        lax.fori_loop(0, T // 16, init_m, 0)

        def step(t, carry):
            ss, si = carry
            base = pl.multiple_of((t // 16) * 16, 16)
            blk = sc_v[pl.ds(base, 16)]
            s_t = jnp.min(jnp.where(lanes == (t % 16), blk, jnp.inf))
            mn = jnp.min(ss)
            do_ins = s_t > mn
            evict_idx = jnp.max(jnp.where(ss == mn, si, jnp.int32(-(2**31))))
            lane = (ss == mn) & (si == evict_idx)
            write_idx = jnp.where(do_ins, evict_idx, t)
            skip = do_ins & (mn == -jnp.inf)
            plsc.store_scatter(
                m_v,
                [jnp.full((16,), write_idx, jnp.int32)],
                jnp.full((16,), t, jnp.int32),
                mask=(lanes == 0) & jnp.logical_not(skip),
            )
            ins_lane = do_ins & lane
            new_ss = jnp.where(ins_lane, jnp.full((16,), s_t), ss)
            new_si = jnp.where(ins_lane, jnp.full((16,), t, jnp.int32), si)
            return new_ss, new_si

        lax.fori_loop(
            0,
            T,
            step,
            (jnp.full((16,), -jnp.inf), lanes - 16),
        )
        pltpu.sync_copy(m_v, m_hbm)


def _sc_scan(scores):
    mesh = plsc.VectorSubcoreMesh(core_axis_name="c", subcore_axis_name="s")
    return pl.kernel(
        _sc_scan_body,
        mesh=mesh,
        compiler_params=pltpu.CompilerParams(needs_layout_passes=False),
        out_type=jax.ShapeDtypeStruct((T,), jnp.int32),
        scratch_types=[
            pltpu.VMEM((T,), jnp.float32),
            pltpu.VMEM((T,), jnp.int32),
            pltpu.SemaphoreType.DMA,
        ],
    )(scores)


def _attn_body(q_ref, k_ref, v_ref, m_ref, o_ref):
    tb = pl.program_id(0)
    t = tb * TB + lax.broadcasted_iota(jnp.int32, (TB, T), 0)
    j = lax.broadcasted_iota(jnp.int32, (TB, T), 1)
    active = (j <= t) & (t < m_ref[...])
    tf = t.astype(jnp.float32)
    boost = jnp.where(j == t, jnp.log(jnp.maximum(KSEL - tf, 1.0)), 0.0)
    add = jnp.where(active, boost, -jnp.inf)
    scale = 1.0 / math.sqrt(HS)
    for h in range(N_HEAD):
        sl = slice(h * HS, (h + 1) * HS)
        logits = jnp.dot(
            q_ref[:, sl], k_ref[:, sl].T, preferred_element_type=jnp.float32
        ) * scale
        masked = logits + add
        mx = jnp.max(masked, axis=1, keepdims=True)
        p = jnp.exp(masked - mx)
        s = jnp.sum(p, axis=1, keepdims=True)
        o_ref[:, sl] = (
            jnp.dot(p, v_ref[:, sl], preferred_element_type=jnp.float32) / s
        )


def _tc_attn(q2d, k2d, v2d, m_row):
    return pl.pallas_call(
        _attn_body,
        grid=(NTB,),
        in_specs=[
            pl.BlockSpec((TB, N_EMBD), lambda tb: (tb, 0)),
            pl.BlockSpec((T, N_EMBD), lambda tb: (0, 0)),
            pl.BlockSpec((T, N_EMBD), lambda tb: (0, 0)),
            pl.BlockSpec((1, T), lambda tb: (0, 0)),
        ],
        out_specs=pl.BlockSpec((TB, N_EMBD), lambda tb: (tb, 0)),
        out_shape=jax.ShapeDtypeStruct((T, N_EMBD), jnp.float32),
    )(q2d, k2d, v2d, m_row)


def _outproj_body(y_ref, w_ref, b_ref, o_ref):
    o_ref[...] = (
        jnp.dot(y_ref[...], w_ref[...], preferred_element_type=jnp.float32)
        + b_ref[...]
    )


def _tc_outproj(y2d, w, b):
    return pl.pallas_call(
        _outproj_body,
        grid=(NTB,),
        in_specs=[
            pl.BlockSpec((TB, N_EMBD), lambda i: (i, 0)),
            pl.BlockSpec((N_EMBD, N_EMBD), lambda i: (0, 0)),
            pl.BlockSpec((1, N_EMBD), lambda i: (0, 0)),
        ],
        out_specs=pl.BlockSpec((TB, N_EMBD), lambda i: (i, 0)),
        out_shape=jax.ShapeDtypeStruct((T, N_EMBD), jnp.float32),
    )(y2d, w, b)


def kernel(x, q_w, q_b, k_w, k_b, v_w, v_b, c_w, c_b, qs_w, qs_b, ks_w, ks_b):
    x2d = x[0]
    ks_col = jnp.zeros((N_EMBD, 128), jnp.float32).at[:, 0].set(ks_w[0])
    w = jnp.concatenate([q_w.T, k_w.T, v_w.T, ks_col], axis=1)
    b = jnp.concatenate(
        [q_b, k_b, v_b, jnp.full((128,), ks_b[0], jnp.float32)]
    )[None, :]
    proj = _tc_proj(x2d, w, b)
    q2d = proj[:, :N_EMBD]
    k2d = proj[:, N_EMBD : 2 * N_EMBD]
    v2d = proj[:, 2 * N_EMBD : 3 * N_EMBD]
    scores = proj[:, 3 * N_EMBD]
    m = _sc_scan(scores)
    y2d = _tc_attn(q2d, k2d, v2d, m[None, :])
    out = _tc_outproj(y2d, c_w.T, c_b[None, :])
    return out[None]


# speedup vs baseline: 6343.9936x; 1.1605x over previous
import math

import jax
import jax.numpy as jnp
from jax import lax
from jax.experimental import pallas as pl
from jax.experimental.pallas import tpu as pltpu
from jax.experimental.pallas import tpu_sc as plsc

N_EMBD = 768
N_HEAD = 12
HS = N_EMBD // N_HEAD
KSEL = 16
T = 2048

TB = 256
NTB = T // TB


def _score_body(x_ref, w_ref, b_ref, o_ref):
    o_ref[...] = (
        lax.dot_general(
            w_ref[...], x_ref[...], (((1,), (1,)), ((), ())),
            preferred_element_type=jnp.float32,
        )
        + b_ref[0, 0]
    )


def _tc_scores(x2d, ks_w, ks_b):
    return pl.pallas_call(
        _score_body,
        in_specs=[
            pl.BlockSpec((T, N_EMBD), lambda: (0, 0)),
            pl.BlockSpec((1, N_EMBD), lambda: (0, 0)),
            pl.BlockSpec((1, 1), lambda: (0, 0), memory_space=pltpu.SMEM),
        ],
        out_specs=pl.BlockSpec((1, T), lambda: (0, 0)),
        out_shape=jax.ShapeDtypeStruct((1, T), jnp.float32),
    )(x2d, ks_w, ks_b[None, :])


def _sc_scan_body(scores_hbm, m_hbm, sc_v, m_v, sem):
    cid = lax.axis_index("c")
    sid = lax.axis_index("s")

    @pl.when((cid == 0) & (sid == 0))
    def _():
        pltpu.sync_copy(scores_hbm, sc_v)

        lanes = lax.iota(jnp.int32, 16)

        def init_m(i, carry):
            m_v[pl.ds(pl.multiple_of(i * 16, 16), 16)] = jnp.full(
                (16,), T, jnp.int32
            )
            return carry

        lax.fori_loop(0, T // 16, init_m, 0)

        def step(t, carry):
            ss, si = carry
            base = pl.multiple_of((t // 16) * 16, 16)
            blk = sc_v[pl.ds(base, 16)]
            s_t = jnp.min(jnp.where(lanes == (t % 16), blk, jnp.inf))
            mn = jnp.min(ss)
            do_ins = s_t > mn
            evict_idx = jnp.max(jnp.where(ss == mn, si, jnp.int32(-(2**31))))
            lane = (ss == mn) & (si == evict_idx)
            write_idx = jnp.where(do_ins, evict_idx, t)
            skip = do_ins & (mn == -jnp.inf)
            plsc.store_scatter(
                m_v,
                [jnp.full((16,), write_idx, jnp.int32)],
                jnp.full((16,), t, jnp.int32),
                mask=(lanes == 0) & jnp.logical_not(skip),
            )
            ins_lane = do_ins & lane
            new_ss = jnp.where(ins_lane, jnp.full((16,), s_t), ss)
            new_si = jnp.where(ins_lane, jnp.full((16,), t, jnp.int32), si)
            return new_ss, new_si

        lax.fori_loop(
            0,
            T,
            step,
            (jnp.full((16,), -jnp.inf), lanes - 16),
        )
        pltpu.sync_copy(m_v, m_hbm)


def _sc_scan(scores):
    mesh = plsc.VectorSubcoreMesh(core_axis_name="c", subcore_axis_name="s")
    return pl.kernel(
        _sc_scan_body,
        mesh=mesh,
        compiler_params=pltpu.CompilerParams(needs_layout_passes=False),
        out_type=jax.ShapeDtypeStruct((T,), jnp.int32),
        scratch_types=[
            pltpu.VMEM((T,), jnp.float32),
            pltpu.VMEM((T,), jnp.int32),
            pltpu.SemaphoreType.DMA,
        ],
    )(scores)


def _proj_body(x_ref, qw_ref, kw_ref, vw_ref, qb_ref, kb_ref, vb_ref,
               q_ref, k_ref, v_ref):
    xb = x_ref[...]
    for w_ref, b_ref, o_ref in (
        (qw_ref, qb_ref, q_ref),
        (kw_ref, kb_ref, k_ref),
        (vw_ref, vb_ref, v_ref),
    ):
        o_ref[...] = (
            lax.dot_general(
                xb, w_ref[...], (((1,), (1,)), ((), ())),
                preferred_element_type=jnp.float32,
            )
            + b_ref[...]
        )


def _tc_proj(x2d, q_w, k_w, v_w, q_b, k_b, v_b):
    wspec = pl.BlockSpec((N_EMBD, N_EMBD), lambda i: (0, 0))
    bspec = pl.BlockSpec((1, N_EMBD), lambda i: (0, 0))
    ospec = pl.BlockSpec((TB, N_EMBD), lambda i: (i, 0))
    return pl.pallas_call(
        _proj_body,
        grid=(NTB,),
        in_specs=[
            pl.BlockSpec((TB, N_EMBD), lambda i: (i, 0)),
            wspec, wspec, wspec, bspec, bspec, bspec,
        ],
        out_specs=[ospec, ospec, ospec],
        out_shape=[jax.ShapeDtypeStruct((T, N_EMBD), jnp.float32)] * 3,
    )(x2d, q_w, k_w, v_w, q_b[None, :], k_b[None, :], v_b[None, :])


def _attn_body(q_ref, k_ref, v_ref, m_ref, cw_ref, cb_ref, o_ref, y_ref):
    tb = pl.program_id(0)
    t = tb * TB + lax.broadcasted_iota(jnp.int32, (TB, T), 0)
    j = lax.broadcasted_iota(jnp.int32, (TB, T), 1)
    active = (j <= t) & (t < m_ref[...])
    tf = t.astype(jnp.float32)
    boost = jnp.where(j == t, jnp.log(jnp.maximum(KSEL - tf, 1.0)), 0.0)
    add = jnp.where(active, boost, -jnp.inf)
    scale = 1.0 / math.sqrt(HS)
    for h in range(N_HEAD):
        sl = slice(h * HS, (h + 1) * HS)
        logits = lax.dot_general(
            q_ref[:, sl], k_ref[:, sl], (((1,), (1,)), ((), ())),
            preferred_element_type=jnp.float32,
        ) * scale
        masked = logits + add
        mx = jnp.max(masked, axis=1, keepdims=True)
        p = jnp.exp(masked - mx)
        s = jnp.sum(p, axis=1, keepdims=True)
        y_ref[:, sl] = (
            jnp.dot(p, v_ref[:, sl], preferred_element_type=jnp.float32) / s
        )
    o_ref[...] = (
        lax.dot_general(
            y_ref[...], cw_ref[...], (((1,), (1,)), ((), ())),
            preferred_element_type=jnp.float32,
        )
        + cb_ref[...]
    )


def _tc_attn(q2d, k2d, v2d, m_row, c_w, c_b):
    return pl.pallas_call(
        _attn_body,
        grid=(NTB,),
        in_specs=[
            pl.BlockSpec((TB, N_EMBD), lambda tb: (tb, 0)),
            pl.BlockSpec((T, N_EMBD), lambda tb: (0, 0)),
            pl.BlockSpec((T, N_EMBD), lambda tb: (0, 0)),
            pl.BlockSpec((1, T), lambda tb: (0, 0)),
            pl.BlockSpec((N_EMBD, N_EMBD), lambda tb: (0, 0)),
            pl.BlockSpec((1, N_EMBD), lambda tb: (0, 0)),
        ],
        out_specs=pl.BlockSpec((TB, N_EMBD), lambda tb: (tb, 0)),
        out_shape=jax.ShapeDtypeStruct((T, N_EMBD), jnp.float32),
        scratch_shapes=[pltpu.VMEM((TB, N_EMBD), jnp.float32)],
    )(q2d, k2d, v2d, m_row, c_w, c_b[None, :])


def kernel(x, q_w, q_b, k_w, k_b, v_w, v_b, c_w, c_b, qs_w, qs_b, ks_w, ks_b):
    x2d = x[0]
    scores = _tc_scores(x2d, ks_w, ks_b)
    m = _sc_scan(scores.reshape(T))
    q2d, k2d, v2d = _tc_proj(x2d, q_w, k_w, v_w, q_b, k_b, v_b)
    out = _tc_attn(q2d, k2d, v2d, m[None, :], c_w, c_b)
    return out[None]
